# Initial kernel scaffold; baseline (speedup 1.0000x reference)
import jax
import jax.numpy as jnp
from jax.experimental import pallas as pl


def kernel(image_features, W1, b1, gamma, beta, W2, b2):
    raise NotImplementedError("write your pallas kernel here")


# speedup vs baseline: 1.5325x; 1.5325x over previous
import functools

import jax
import jax.numpy as jnp
from jax.experimental import pallas as pl
from jax.experimental.pallas import tpu as pltpu

_B, _N, _C, _H, _W = 8, 6, 128, 64, 176
_BN = _B * _N
_P = _H * _W
_HID = 64
_NC = 4
_K = 1000
_EPS = 1e-5
_TAPS = 9
_INV_COUNT = 1.0 / (_BN * _P)

_PREC = jax.lax.Precision.DEFAULT


def _conv_stats_body(x_ref, ws_ref, b1_ref, y_ref, s_ref, ss_ref, pall):
    i = pl.program_id(0)
    x = x_ref[0]
    pall[:, :] = jax.lax.dot_general(
        ws_ref[:, :], x, (((1,), (0,)), ((), ())),
        precision=_PREC, preferred_element_type=jnp.float32)

    wc = jax.lax.broadcasted_iota(jnp.int32, (1, _P), 1) % _W
    m_left = (wc > 0).astype(jnp.float32)
    m_right = (wc < _W - 1).astype(jnp.float32)

    y_ref[0, :, :] = pall[4 * _HID:5 * _HID, :] + b1_ref[:, :]
    for t in range(_TAPS):
        if t == 4:
            continue
        dh, dw = t // 3, t % 3
        off = (dh - 1) * _W + (dw - 1)
        a = max(0, -off)
        b = _P - max(0, off)
        src = pall[t * _HID:(t + 1) * _HID, a + off:b + off]
        if dw == 0:
            src = src * m_left[:, a:b]
        elif dw == 2:
            src = src * m_right[:, a:b]
        y_ref[0, :, a:b] += src

    yv = y_ref[0]
    s = jnp.sum(yv, axis=1, keepdims=True)
    ss = jnp.sum(yv * yv, axis=1, keepdims=True)

    @pl.when(i == 0)
    def _():
        s_ref[:, :] = s
        ss_ref[:, :] = ss

    @pl.when(i > 0)
    def _():
        s_ref[:, :] += s
        ss_ref[:, :] += ss


def _head_body(y_ref, s_ref, ss_ref, g_ref, be_ref, w2_ref, b2_ref,
               probs_ref, mask_ref):
    mu = s_ref[:, :] * _INV_COUNT
    var = ss_ref[:, :] * _INV_COUNT - mu * mu
    scale = g_ref[:, :] / jnp.sqrt(var + _EPS)
    shift = be_ref[:, :] - mu * scale
    yn = jnp.maximum(y_ref[0] * scale + shift, 0.0)
    logits = jax.lax.dot_general(
        w2_ref[:, :], yn, (((1,), (0,)), ((), ())),
        precision=_PREC, preferred_element_type=jnp.float32) + b2_ref[:, :]
    mx = jnp.max(logits, axis=0, keepdims=True)
    e = jnp.exp(logits - mx)
    probs = e / jnp.sum(e, axis=0, keepdims=True)
    probs_ref[0] = probs

    scores = jnp.max(probs[1:_NC, :], axis=0, keepdims=True)
    nonempty = scores > probs[0:1, :]

    si = jax.lax.bitcast_convert_type(scores, jnp.int32)
    t = jnp.int32(0)
    for bit in range(30, -1, -1):
        cand = t | jnp.int32(1 << bit)
        cnt = jnp.sum((si >= cand).astype(jnp.int32))
        t = jnp.where(cnt >= _K, cand, t)
    keep = (si >= t) & nonempty
    mask_ref[0] = keep.astype(jnp.float32)


@jax.jit
def kernel(image_features, W1, b1, gamma, beta, W2, b2):
    x = image_features.reshape(_BN, _C, _P)
    ws = jnp.transpose(W1, (2, 3, 0, 1)).reshape(_TAPS * _HID, _C)
    b1c = b1.reshape(_HID, 1)
    gc = gamma.reshape(_HID, 1)
    bec = beta.reshape(_HID, 1)
    w2m = W2.reshape(_NC, _HID)
    b2c = b2.reshape(_NC, 1)

    f32 = jnp.float32
    y, s, ss = pl.pallas_call(
        _conv_stats_body,
        grid=(_BN,),
        in_specs=[
            pl.BlockSpec((1, _C, _P), lambda i: (i, 0, 0)),
            pl.BlockSpec((_TAPS * _HID, _C), lambda i: (0, 0)),
            pl.BlockSpec((_HID, 1), lambda i: (0, 0)),
        ],
        out_specs=[
            pl.BlockSpec((1, _HID, _P), lambda i: (i, 0, 0)),
            pl.BlockSpec((_HID, 1), lambda i: (0, 0)),
            pl.BlockSpec((_HID, 1), lambda i: (0, 0)),
        ],
        out_shape=[
            jax.ShapeDtypeStruct((_BN, _HID, _P), f32),
            jax.ShapeDtypeStruct((_HID, 1), f32),
            jax.ShapeDtypeStruct((_HID, 1), f32),
        ],
        scratch_shapes=[pltpu.VMEM((_TAPS * _HID, _P), f32)],
    )(x, ws, b1c)

    probs, mask = pl.pallas_call(
        _head_body,
        grid=(_BN,),
        in_specs=[
            pl.BlockSpec((1, _HID, _P), lambda i: (i, 0, 0)),
            pl.BlockSpec((_HID, 1), lambda i: (0, 0)),
            pl.BlockSpec((_HID, 1), lambda i: (0, 0)),
            pl.BlockSpec((_HID, 1), lambda i: (0, 0)),
            pl.BlockSpec((_HID, 1), lambda i: (0, 0)),
            pl.BlockSpec((_NC, _HID), lambda i: (0, 0)),
            pl.BlockSpec((_NC, 1), lambda i: (0, 0)),
        ],
        out_specs=[
            pl.BlockSpec((1, _NC, _P), lambda i: (i, 0, 0)),
            pl.BlockSpec((1, 1, _P), lambda i: (i, 0, 0)),
        ],
        out_shape=[
            jax.ShapeDtypeStruct((_BN, _NC, _P), f32),
            jax.ShapeDtypeStruct((_BN, 1, _P), f32),
        ],
    )(y, s, ss, gc, bec, w2m, b2c)

    probs_out = probs.reshape(_B, _N, _NC, _H, _W)
    mask_out = mask.reshape(_B, _N, _H, _W).astype(jnp.bool_)
    return probs_out, mask_out


# speedup vs baseline: 1.9744x; 1.2884x over previous
import jax
import jax.numpy as jnp
from jax.experimental import pallas as pl
from jax.experimental.pallas import tpu as pltpu

_B, _N, _C, _H, _W = 8, 6, 128, 64, 176
_BN = _B * _N
_P = _H * _W
_HID = 64
_NC = 4
_K = 1000
_EPS = 1e-5
_TAPS = 9
_INV_COUNT = 1.0 / (_BN * _P)

_PREC = jax.lax.Precision.DEFAULT


def _conv_stats_body(x_ref, ws_ref, b1_ref, y_ref, s_ref, ss_ref, pall):
    i = pl.program_id(0)
    x = x_ref[0]
    pall[:, :] = jax.lax.dot_general(
        ws_ref[:, :], x, (((1,), (0,)), ((), ())),
        precision=_PREC, preferred_element_type=jnp.float32)

    wc = jax.lax.broadcasted_iota(jnp.int32, (1, _P), 1) % _W
    m_left = (wc > 0).astype(jnp.float32)
    m_right = (wc < _W - 1).astype(jnp.float32)

    y_ref[0, :, :] = pall[4 * _HID:5 * _HID, :] + b1_ref[:, :]
    for t in range(_TAPS):
        if t == 4:
            continue
        dh, dw = t // 3, t % 3
        off = (dh - 1) * _W + (dw - 1)
        a = max(0, -off)
        b = _P - max(0, off)
        src = pall[t * _HID:(t + 1) * _HID, a + off:b + off]
        if dw == 0:
            src = src * m_left[:, a:b]
        elif dw == 2:
            src = src * m_right[:, a:b]
        y_ref[0, :, a:b] += src

    yv = y_ref[0]
    s = jnp.sum(yv, axis=1, keepdims=True)
    ss = jnp.sum(yv * yv, axis=1, keepdims=True)

    @pl.when(i == 0)
    def _():
        s_ref[:, :] = s
        ss_ref[:, :] = ss

    @pl.when(i > 0)
    def _():
        s_ref[:, :] += s
        ss_ref[:, :] += ss


def _head_body(y_ref, s_ref, ss_ref, g_ref, be_ref, w2_ref, b2_ref,
               probs_ref, sp_ref):
    mu = s_ref[:, :] * _INV_COUNT
    var = ss_ref[:, :] * _INV_COUNT - mu * mu
    scale = g_ref[:, :] / jnp.sqrt(var + _EPS)
    shift = be_ref[:, :] - mu * scale
    yn = jnp.maximum(y_ref[0] * scale + shift, 0.0)
    logits = jax.lax.dot_general(
        w2_ref[:, :], yn, (((1,), (0,)), ((), ())),
        precision=_PREC, preferred_element_type=jnp.float32) + b2_ref[:, :]
    mx = jnp.max(logits, axis=0, keepdims=True)
    e = jnp.exp(logits - mx)
    probs = e / jnp.sum(e, axis=0, keepdims=True)
    probs_ref[0] = probs

    scores = jnp.max(probs[1:_NC, :], axis=0, keepdims=True)
    nonempty = scores > probs[0:1, :]
    sp_ref[0] = jnp.where(nonempty, -scores, scores)


def _select_body(sp_ref, mask_ref):
    sp = sp_ref[:, :]
    si = jax.lax.bitcast_convert_type(sp, jnp.int32)
    nonempty = si < 0
    sabs = si & jnp.int32(0x7FFFFFFF)
    t = jnp.zeros((_BN, 1), jnp.int32)
    for bit in range(30, -1, -1):
        cand = t | jnp.int32(1 << bit)
        cnt = jnp.sum((sabs >= cand).astype(jnp.int32), axis=1, keepdims=True)
        t = jnp.where(cnt >= _K, cand, t)
    keep = (sabs >= t) & nonempty
    mask_ref[:, :] = keep.astype(jnp.float32)


@jax.jit
def kernel(image_features, W1, b1, gamma, beta, W2, b2):
    x = image_features.astype(jnp.bfloat16).reshape(_BN, _C, _P)
    ws = jnp.transpose(W1, (2, 3, 0, 1)).reshape(_TAPS * _HID, _C)
    ws = ws.astype(jnp.bfloat16)
    b1c = b1.reshape(_HID, 1)
    gc = gamma.reshape(_HID, 1)
    bec = beta.reshape(_HID, 1)
    w2m = W2.reshape(_NC, _HID)
    b2c = b2.reshape(_NC, 1)

    f32 = jnp.float32
    y, s, ss = pl.pallas_call(
        _conv_stats_body,
        grid=(_BN,),
        in_specs=[
            pl.BlockSpec((1, _C, _P), lambda i: (i, 0, 0)),
            pl.BlockSpec((_TAPS * _HID, _C), lambda i: (0, 0)),
            pl.BlockSpec((_HID, 1), lambda i: (0, 0)),
        ],
        out_specs=[
            pl.BlockSpec((1, _HID, _P), lambda i: (i, 0, 0)),
            pl.BlockSpec((_HID, 1), lambda i: (0, 0)),
            pl.BlockSpec((_HID, 1), lambda i: (0, 0)),
        ],
        out_shape=[
            jax.ShapeDtypeStruct((_BN, _HID, _P), f32),
            jax.ShapeDtypeStruct((_HID, 1), f32),
            jax.ShapeDtypeStruct((_HID, 1), f32),
        ],
        scratch_shapes=[pltpu.VMEM((_TAPS * _HID, _P), f32)],
    )(x, ws, b1c)

    probs, spacked = pl.pallas_call(
        _head_body,
        grid=(_BN,),
        in_specs=[
            pl.BlockSpec((1, _HID, _P), lambda i: (i, 0, 0)),
            pl.BlockSpec((_HID, 1), lambda i: (0, 0)),
            pl.BlockSpec((_HID, 1), lambda i: (0, 0)),
            pl.BlockSpec((_HID, 1), lambda i: (0, 0)),
            pl.BlockSpec((_HID, 1), lambda i: (0, 0)),
            pl.BlockSpec((_NC, _HID), lambda i: (0, 0)),
            pl.BlockSpec((_NC, 1), lambda i: (0, 0)),
        ],
        out_specs=[
            pl.BlockSpec((1, _NC, _P), lambda i: (i, 0, 0)),
            pl.BlockSpec((1, 1, _P), lambda i: (i, 0, 0)),
        ],
        out_shape=[
            jax.ShapeDtypeStruct((_BN, _NC, _P), f32),
            jax.ShapeDtypeStruct((_BN, 1, _P), f32),
        ],
    )(y, s, ss, gc, bec, w2m, b2c)

    mask = pl.pallas_call(
        _select_body,
        grid=(1,),
        in_specs=[pl.BlockSpec((_BN, _P), lambda i: (0, 0))],
        out_specs=pl.BlockSpec((_BN, _P), lambda i: (0, 0)),
        out_shape=jax.ShapeDtypeStruct((_BN, _P), f32),
    )(spacked.reshape(_BN, _P))

    probs_out = probs.reshape(_B, _N, _NC, _H, _W)
    mask_out = mask.reshape(_B, _N, _H, _W).astype(jnp.bool_)
    return probs_out, mask_out


# speedup vs baseline: 2.4012x; 1.2161x over previous
import jax
import jax.numpy as jnp
from jax.experimental import pallas as pl
from jax.experimental.pallas import tpu as pltpu

_B, _N, _C, _H, _W = 8, 6, 128, 64, 176
_BN = _B * _N
_P = _H * _W
_HID = 64
_NC = 4
_K = 1000
_EPS = 1e-5
_TAPS = 9
_INV_COUNT = 1.0 / (_BN * _P)

_PREC = jax.lax.Precision.DEFAULT


def _conv_stats_body(x_ref, wd_ref, b1_ref, y_ref, s_ref, ss_ref, x3, q):
    i = pl.program_id(0)
    x = x_ref[0]
    wc = jax.lax.broadcasted_iota(jnp.int32, (1, _P), 1) % _W
    zcol = jnp.zeros((_C, 1), jnp.bfloat16)
    xm = jnp.concatenate([zcol, x[:, :_P - 1]], axis=1)
    xm = jnp.where(wc > 0, xm, jnp.bfloat16(0))
    xp = jnp.concatenate([x[:, 1:], zcol], axis=1)
    xp = jnp.where(wc < _W - 1, xp, jnp.bfloat16(0))
    x3[0:_C, :] = xm
    x3[_C:2 * _C, :] = x
    x3[2 * _C:3 * _C, :] = xp
    q[:, :] = jax.lax.dot_general(
        wd_ref[:, :], x3[:, :], (((1,), (0,)), ((), ())),
        precision=_PREC, preferred_element_type=jnp.float32)

    y_ref[0, :, :] = q[_HID:2 * _HID, :] + b1_ref[:, :]
    y_ref[0, :, _W:_P] += q[0:_HID, 0:_P - _W]
    y_ref[0, :, 0:_P - _W] += q[2 * _HID:3 * _HID, _W:_P]

    yv = y_ref[0]
    s = jnp.sum(yv, axis=1, keepdims=True)
    ss = jnp.sum(yv * yv, axis=1, keepdims=True)

    @pl.when(i == 0)
    def _():
        s_ref[:, :] = s
        ss_ref[:, :] = ss

    @pl.when(i > 0)
    def _():
        s_ref[:, :] += s
        ss_ref[:, :] += ss


def _head_body(y_ref, s_ref, ss_ref, g_ref, be_ref, w2_ref, b2_ref,
               probs_ref, sp_ref):
    mu = s_ref[:, :] * _INV_COUNT
    var = ss_ref[:, :] * _INV_COUNT - mu * mu
    scale = g_ref[:, :] / jnp.sqrt(var + _EPS)
    shift = be_ref[:, :] - mu * scale
    yn = jnp.maximum(y_ref[0] * scale + shift, 0.0)
    logits = jax.lax.dot_general(
        w2_ref[:, :], yn, (((1,), (0,)), ((), ())),
        precision=_PREC, preferred_element_type=jnp.float32) + b2_ref[:, :]
    mx = jnp.max(logits, axis=0, keepdims=True)
    e = jnp.exp(logits - mx)
    probs = e / jnp.sum(e, axis=0, keepdims=True)
    probs_ref[0] = probs

    scores = jnp.max(probs[1:_NC, :], axis=0, keepdims=True)
    nonempty = scores > probs[0:1, :]
    sp_ref[0] = jnp.where(nonempty, -scores, scores)


def _select_body(sp_ref, mask_ref):
    sp = sp_ref[:, :]
    si = jax.lax.bitcast_convert_type(sp, jnp.int32)
    nonempty = si < 0
    sabs = si & jnp.int32(0x7FFFFFFF)
    t = jnp.zeros((_BN, 1), jnp.int32)
    for bit in range(30, -1, -1):
        cand = t | jnp.int32(1 << bit)
        cnt = jnp.sum((sabs >= cand).astype(jnp.int32), axis=1, keepdims=True)
        t = jnp.where(cnt >= _K, cand, t)
    keep = (sabs >= t) & nonempty
    mask_ref[:, :] = keep.astype(jnp.float32)


@jax.jit
def kernel(image_features, W1, b1, gamma, beta, W2, b2):
    x = image_features.astype(jnp.bfloat16).reshape(_BN, _C, _P)
    wd = jnp.transpose(W1, (2, 0, 3, 1)).reshape(3 * _HID, 3 * _C)
    wd = wd.astype(jnp.bfloat16)
    b1c = b1.reshape(_HID, 1)
    gc = gamma.reshape(_HID, 1)
    bec = beta.reshape(_HID, 1)
    w2m = W2.reshape(_NC, _HID)
    b2c = b2.reshape(_NC, 1)

    f32 = jnp.float32
    y, s, ss = pl.pallas_call(
        _conv_stats_body,
        grid=(_BN,),
        in_specs=[
            pl.BlockSpec((1, _C, _P), lambda i: (i, 0, 0)),
            pl.BlockSpec((3 * _HID, 3 * _C), lambda i: (0, 0)),
            pl.BlockSpec((_HID, 1), lambda i: (0, 0)),
        ],
        out_specs=[
            pl.BlockSpec((1, _HID, _P), lambda i: (i, 0, 0)),
            pl.BlockSpec((_HID, 1), lambda i: (0, 0)),
            pl.BlockSpec((_HID, 1), lambda i: (0, 0)),
        ],
        out_shape=[
            jax.ShapeDtypeStruct((_BN, _HID, _P), f32),
            jax.ShapeDtypeStruct((_HID, 1), f32),
            jax.ShapeDtypeStruct((_HID, 1), f32),
        ],
        scratch_shapes=[pltpu.VMEM((3 * _C, _P), jnp.bfloat16),
                        pltpu.VMEM((3 * _HID, _P), f32)],
    )(x, wd, b1c)

    probs, spacked = pl.pallas_call(
        _head_body,
        grid=(_BN,),
        in_specs=[
            pl.BlockSpec((1, _HID, _P), lambda i: (i, 0, 0)),
            pl.BlockSpec((_HID, 1), lambda i: (0, 0)),
            pl.BlockSpec((_HID, 1), lambda i: (0, 0)),
            pl.BlockSpec((_HID, 1), lambda i: (0, 0)),
            pl.BlockSpec((_HID, 1), lambda i: (0, 0)),
            pl.BlockSpec((_NC, _HID), lambda i: (0, 0)),
            pl.BlockSpec((_NC, 1), lambda i: (0, 0)),
        ],
        out_specs=[
            pl.BlockSpec((1, _NC, _P), lambda i: (i, 0, 0)),
            pl.BlockSpec((1, 1, _P), lambda i: (i, 0, 0)),
        ],
        out_shape=[
            jax.ShapeDtypeStruct((_BN, _NC, _P), f32),
            jax.ShapeDtypeStruct((_BN, 1, _P), f32),
        ],
    )(y, s, ss, gc, bec, w2m, b2c)

    mask = pl.pallas_call(
        _select_body,
        grid=(1,),
        in_specs=[pl.BlockSpec((_BN, _P), lambda i: (0, 0))],
        out_specs=pl.BlockSpec((_BN, _P), lambda i: (0, 0)),
        out_shape=jax.ShapeDtypeStruct((_BN, _P), f32),
    )(spacked.reshape(_BN, _P))

    probs_out = probs.reshape(_B, _N, _NC, _H, _W)
    mask_out = mask.reshape(_B, _N, _H, _W).astype(jnp.bool_)
    return probs_out, mask_out
